# Initial kernel scaffold; baseline (speedup 1.0000x reference)
import jax
import jax.numpy as jnp
from jax.experimental import pallas as pl


def kernel(z, x, We1, be1, We2, be2, Wc, Wn1, bn1, Wn2, bn2, rbf_centers, rbf_gamma, Wq, bq, Wk, bk, Wv, bv, Wo, bo, Wa, ba):
    raise NotImplementedError("write your pallas kernel here")


# speedup vs baseline: 2.4945x; 2.4945x over previous
import functools
import math

import jax
import jax.numpy as jnp
from jax.experimental import pallas as pl
from jax.experimental.pallas import tpu as pltpu

_HIGH = jax.lax.Precision.HIGHEST


def _bf(v):
    return v.astype(jnp.bfloat16).astype(jnp.float32)


def _silu(v):
    return v * jax.nn.sigmoid(v)


def _body(z_ref, x_ref,
          We1a_ref, We1b_ref, wd2_ref, be1_ref, We2_ref, be2_ref, Wc_ref,
          Wn1a_ref, Wn1b_ref, bn1_ref, Wn2_ref, bn2_ref,
          cent_ref, gam_ref,
          Wqr_ref, Wqh_ref, bq_ref,
          Wkr_ref, Wkh_ref, bk_ref,
          Wvr_ref, Wvh_ref, bv_ref,
          Wo_ref, bo_ref, Wa_ref, ba_ref,
          out_ref, *, depth, n, inner):
    h = z_ref[0]
    xc = x_ref[0]
    inv_n = 1.0 / float(n)

    for l in range(depth):
        xT = xc.T
        d2 = ((xc[:, 0:1] - xT[0:1, :]) ** 2
              + (xc[:, 1:2] - xT[1:2, :]) ** 2
              + (xc[:, 2:3] - xT[2:3, :]) ** 2)

        A = jnp.dot(h, We1a_ref[l]) + be1_ref[l]
        Bm = jnp.dot(h, We1b_ref[l])
        m = (A[:, None, :] + Bm[None, :, :]
             + _bf(d2)[:, :, None] * _bf(wd2_ref[l])[None])
        m = _silu(m)
        m = jax.lax.dot_general(m, We2_ref[l],
                                (((2,), (0,)), ((), ()))) + be2_ref[l][None]
        m = _silu(m)

        cw = jnp.sum(m * Wc_ref[l][None], axis=2)
        sden = jnp.sqrt(d2) + 1.0
        lane = jax.lax.broadcasted_iota(jnp.int32, (1, 128), 1)
        delta = jnp.zeros_like(xc)
        for c in range(3):
            relc = xc[:, c:c + 1] - xT[c:c + 1, :]
            dc = jnp.mean(relc / sden * cw, axis=1, keepdims=True)
            delta = delta + jnp.where(lane == c, dc, 0.0)
        xc = xc + delta

        agg = jnp.sum(m, axis=1)
        hu = _silu(jnp.dot(h, Wn1a_ref[l])
                   + jnp.dot(agg, Wn1b_ref[l])
                   + bn1_ref[l])
        h = h + jnp.dot(hu, Wn2_ref[l]) + bn2_ref[l]

    centpt = jnp.mean(xc, axis=0, keepdims=True)
    dfc = xc - centpt
    d = jnp.sqrt(jnp.sum(dfc * dfc, axis=1, keepdims=True) + 1e-12)
    r = jnp.exp(-gam_ref[...] * (d - cent_ref[...]) ** 2)

    q = jnp.dot(r, Wqr_ref[...]) + jnp.dot(h, Wqh_ref[...]) + bq_ref[...]
    k = jnp.dot(r, Wkr_ref[...]) + jnp.dot(h, Wkh_ref[...]) + bk_ref[...]
    v = jnp.dot(r, Wvr_ref[...]) + jnp.dot(h, Wvh_ref[...]) + bv_ref[...]
    logits = jax.lax.dot_general(q, k, (((1,), (1,)), ((), ())))
    att = jax.nn.softmax(logits * (1.0 / math.sqrt(float(inner))), axis=-1)
    tok = jnp.dot(jnp.dot(att, v), Wo_ref[...]) + bo_ref[...]

    preds = jnp.max(tok, axis=0, keepdims=True)
    val = jnp.sum(preds * Wa_ref[...], keepdims=True)[:, 0:1] + ba_ref[...]
    out_ref[...] = jnp.broadcast_to(val[None], out_ref.shape)


def kernel(z, x, We1, be1, We2, be2, Wc, Wn1, bn1, Wn2, bn2,
           rbf_centers, rbf_gamma, Wq, bq, Wk, bk, Wv, bv, Wo, bo, Wa, ba):
    B, N, DIM = z.shape
    DEPTH, _, HID = We2.shape
    BASIS = rbf_centers.shape[0]
    INNER = DIM + BASIS

    xp = jnp.pad(x, ((0, 0), (0, 0), (0, 128 - x.shape[-1])))

    We1a = We1[:, :DIM, :]
    We1b = We1[:, DIM:2 * DIM, :]
    wd2 = We1[:, 2 * DIM, :][:, None, :]
    be1r = be1[:, None, :]
    be2r = be2[:, None, :]
    Wcr = Wc[:, :, 0][:, None, :]
    Wn1a = Wn1[:, :DIM, :]
    Wn1b = Wn1[:, DIM:, :]
    bn1r = bn1[:, None, :]
    bn2r = bn2[:, None, :]
    centr = rbf_centers[None, :]
    gamr = rbf_gamma[None, :]
    Wa_r = Wa[:, 0][None, :]
    ba_r = ba.reshape(1, 1)

    def full(a):
        return pl.BlockSpec(a.shape, lambda b: (0,) * a.ndim)

    ops = [We1a, We1b, wd2, be1r, We2, be2r, Wcr,
           Wn1a, Wn1b, bn1r, Wn2, bn2r,
           centr, gamr,
           Wq[:BASIS], Wq[BASIS:], bq[None, :],
           Wk[:BASIS], Wk[BASIS:], bk[None, :],
           Wv[:BASIS], Wv[BASIS:], bv[None, :],
           Wo, bo[None, :], Wa_r, ba_r]

    res = pl.pallas_call(
        functools.partial(_body, depth=DEPTH, n=N, inner=INNER),
        grid=(B,),
        in_specs=[pl.BlockSpec((1, N, DIM), lambda b: (b, 0, 0)),
                  pl.BlockSpec((1, N, 128), lambda b: (b, 0, 0))]
                 + [full(a) for a in ops],
        out_specs=pl.BlockSpec((1, 1, 128), lambda b: (b, 0, 0)),
        out_shape=jax.ShapeDtypeStruct((B, 1, 128), jnp.float32),
        compiler_params=pltpu.CompilerParams(
            dimension_semantics=("arbitrary",),
        ),
    )(z, xp, *ops)

    return res[:, 0, 0:1].T
